# baseline (speedup 1.0000x reference)
import functools

import jax
import jax.numpy as jnp
from jax import lax
from jax.experimental import pallas as pl

BATCH = 16384
BLOCK = 2048
TEMP_INV = 1.0 / 5.0


def _mlp_block(x_ref, w1_ref, b1_ref, w2_ref, b2_ref, o_ref):
    x = x_ref[...]
    h = lax.dot_general(x, w1_ref[...], (((1,), (1,)), ((), ())),
                        preferred_element_type=jnp.float32)
    h = jnp.maximum(h + b1_ref[...], 0.0)
    o = lax.dot_general(h, w2_ref[...], (((1,), (1,)), ((), ())),
                        preferred_element_type=jnp.float32)
    o_ref[...] = (o + b2_ref[...]) * TEMP_INV


@jax.jit
def kernel(x, W1, b1, W2, b2):
    grid = (BATCH // BLOCK,)
    return pl.pallas_call(
        _mlp_block,
        grid=grid,
        in_specs=[
            pl.BlockSpec((BLOCK, x.shape[1]), lambda i: (i, 0)),
            pl.BlockSpec(W1.shape, lambda i: (0, 0)),
            pl.BlockSpec((1, b1.shape[0]), lambda i: (0, 0)),
            pl.BlockSpec(W2.shape, lambda i: (0, 0)),
            pl.BlockSpec((1, b2.shape[0]), lambda i: (0, 0)),
        ],
        out_specs=pl.BlockSpec((BLOCK, W2.shape[0]), lambda i: (i, 0)),
        out_shape=jax.ShapeDtypeStruct((BATCH, W2.shape[0]), jnp.float32),
    )(x, W1, b1.reshape(1, -1), W2, b2.reshape(1, -1))


# speedup vs baseline: 1.1403x; 1.1403x over previous
import functools

import jax
import jax.numpy as jnp
from jax import lax
from jax.experimental import pallas as pl

BATCH = 16384
BLOCK = 4096
TEMP_INV = 1.0 / 5.0


def _mlp_block(x_ref, w1_ref, b1_ref, w2_ref, b2_ref, o_ref):
    x = x_ref[...]
    h = lax.dot_general(x, w1_ref[...], (((1,), (1,)), ((), ())),
                        preferred_element_type=jnp.float32)
    h = jnp.maximum(h + b1_ref[...], 0.0)
    o = lax.dot_general(h, w2_ref[...], (((1,), (1,)), ((), ())),
                        preferred_element_type=jnp.float32)
    o_ref[...] = (o + b2_ref[...]) * TEMP_INV


@jax.jit
def kernel(x, W1, b1, W2, b2):
    grid = (BATCH // BLOCK,)
    return pl.pallas_call(
        _mlp_block,
        grid=grid,
        in_specs=[
            pl.BlockSpec((BLOCK, x.shape[1]), lambda i: (i, 0)),
            pl.BlockSpec(W1.shape, lambda i: (0, 0)),
            pl.BlockSpec((1, b1.shape[0]), lambda i: (0, 0)),
            pl.BlockSpec(W2.shape, lambda i: (0, 0)),
            pl.BlockSpec((1, b2.shape[0]), lambda i: (0, 0)),
        ],
        out_specs=pl.BlockSpec((BLOCK, W2.shape[0]), lambda i: (i, 0)),
        out_shape=jax.ShapeDtypeStruct((BATCH, W2.shape[0]), jnp.float32),
    )(x, W1, b1.reshape(1, -1), W2, b2.reshape(1, -1))
